# baseline (speedup 1.0000x reference)
import functools

import jax
import jax.numpy as jnp
from jax import lax
from jax.experimental import pallas as pl
from jax.experimental.pallas import tpu as pltpu
from jax.experimental.pallas import tpu_sc as plsc

M_ROWS = 1000000
DIM = 16
LANES = 16
NUM_CORES = 2
NUM_SUBCORES = 16
NUM_WORKERS = NUM_CORES * NUM_SUBCORES

CHUNK_COLS = 1024
GROUPS = CHUNK_COLS // LANES
CHUNKS_PER_WORKER = 12
SC_ROWS = CHUNK_COLS * NUM_WORKERS * CHUNKS_PER_WORKER
TC_BLOCK = 65536
TC_BLOCK0 = SC_ROWS // TC_BLOCK
NBUF = 4


def _sc_body(itT_hbm, user_hbm, out_hbm, u_v, *bufs):
    bufA = list(bufs[0:NBUF])
    bufB = list(bufs[NBUF:2 * NBUF])
    outv = list(bufs[2 * NBUF:3 * NBUF])
    insem = list(bufs[3 * NBUF:4 * NBUF])
    outsem = list(bufs[4 * NBUF:5 * NBUF])
    wid = lax.axis_index("s") * NUM_CORES + lax.axis_index("c")

    pltpu.sync_copy(user_hbm.at[0], u_v)
    u = u_v[...]
    ub = [
        jnp.take_along_axis(u, jnp.full((LANES,), d, jnp.int32), 0,
                            mode="promise_in_bounds")
        for d in range(DIM)
    ]

    def make_compute(k):
        bA, bB, ov = bufA[k], bufB[k], outv[k]

        def group_body(g, _):
            off = g * LANES
            acc = bA[0, pl.ds(off, LANES)] * ub[0]
            for d in range(1, 8):
                acc = acc + bA[d, pl.ds(off, LANES)] * ub[d]
            for d in range(8, DIM):
                acc = acc + bB[d - 8, pl.ds(off, LANES)] * ub[d]
            ov[pl.ds(off, LANES)] = acc
            return 0
        return lambda: lax.fori_loop(0, GROUPS, group_body, 0, unroll=False)

    computes = [make_compute(k) for k in range(NBUF)]

    def in_slices(c):
        col0 = c * CHUNK_COLS
        return (itT_hbm.at[pl.ds(0, 8), pl.ds(col0, CHUNK_COLS)],
                itT_hbm.at[pl.ds(8, 8), pl.ds(col0, CHUNK_COLS)])

    def start_in(c, k):
        sa, sb = in_slices(c)
        pltpu.async_copy(sa, bufA[k], insem[k])
        pltpu.async_copy(sb, bufB[k], insem[k])

    def wait_in(c, k):
        sa, sb = in_slices(c)
        pltpu.make_async_copy(sa, bufA[k], insem[k]).wait()
        pltpu.make_async_copy(sb, bufB[k], insem[k]).wait()

    def start_out(c, k):
        pltpu.async_copy(
            outv[k], out_hbm.at[pl.ds(c * CHUNK_COLS, CHUNK_COLS)], outsem[k])

    def wait_out(c, k):
        pltpu.make_async_copy(
            outv[k], out_hbm.at[pl.ds(c * CHUNK_COLS, CHUNK_COLS)],
            outsem[k]).wait()

    ncw = CHUNKS_PER_WORKER
    cid = lambda i: wid + i * NUM_WORKERS

    for k in range(NBUF):
        start_in(cid(k), k)
    nquads = ncw // NBUF

    def quad_body(p, _):
        j0 = NBUF * p
        for k in range(NBUF):
            j = j0 + k
            wait_in(cid(j), k)

            @pl.when(p > 0)
            def _():
                wait_out(cid(j - NBUF), k)

            computes[k]()
            start_out(cid(j), k)

            @pl.when(j + NBUF < ncw)
            def _():
                start_in(cid(j + NBUF), k)

        return 0

    lax.fori_loop(0, nquads, quad_body, 0, unroll=False)
    for k in range(NBUF):
        wait_out(0, k)


def _tc_body(x_ref, u_ref, o_ref):
    prod = jax.lax.dot_general(
        u_ref[...], x_ref[...], (((1,), (0,)), ((), ())),
        preferred_element_type=jnp.float32,
    )
    o_ref[...] = prod.reshape(-1)


def kernel(items_emb, user_emb):
    items_t = items_emb.T

    mesh = plsc.VectorSubcoreMesh(
        core_axis_name="c", subcore_axis_name="s",
        num_cores=NUM_CORES, num_subcores=NUM_SUBCORES,
    )
    sc_run = pl.kernel(
        _sc_body,
        out_type=jax.ShapeDtypeStruct((SC_ROWS,), jnp.float32),
        mesh=mesh,
        compiler_params=pltpu.CompilerParams(
            needs_layout_passes=False, use_tc_tiling_on_sc=True,
        ),
        scratch_types=(
            [pltpu.VMEM((LANES,), jnp.float32)]
            + [pltpu.VMEM((8, CHUNK_COLS), jnp.float32)] * NBUF
            + [pltpu.VMEM((8, CHUNK_COLS), jnp.float32)] * NBUF
            + [pltpu.VMEM((CHUNK_COLS,), jnp.float32)] * NBUF
            + [pltpu.SemaphoreType.DMA] * NBUF
            + [pltpu.SemaphoreType.DMA] * NBUF
        ),
    )
    sc_out = sc_run(items_t, user_emb)

    n_tc_blocks = pl.cdiv(M_ROWS, TC_BLOCK) - TC_BLOCK0
    tc_full = pl.pallas_call(
        _tc_body,
        grid=(n_tc_blocks,),
        in_specs=[
            pl.BlockSpec((DIM, TC_BLOCK), lambda i: (0, TC_BLOCK0 + i)),
            pl.BlockSpec((1, DIM), lambda i: (0, 0)),
        ],
        out_specs=pl.BlockSpec((TC_BLOCK,), lambda i: (TC_BLOCK0 + i,)),
        out_shape=jax.ShapeDtypeStruct((M_ROWS,), jnp.float32),
    )(items_t, user_emb)

    return lax.dynamic_update_slice(tc_full, sc_out, (0,))
